# baseline (speedup 1.0000x reference)
import jax
import jax.numpy as jnp
from jax.experimental import pallas as pl
from jax.experimental.pallas import tpu as pltpu

_EMB_D = 768
_BB = 3328


def _body(scalars_ref, pool_ref, x_ref, p_ref, xc_ref):
    tid = scalars_ref[0]
    valid = scalars_ref[1]
    row = pool_ref[pl.ds(tid, 1), :]
    row = jnp.where(valid == 1, row, jnp.full_like(row, jnp.nan))
    p_ref[...] = jnp.broadcast_to(row, p_ref.shape)
    xc_ref[...] = x_ref[...]


def kernel(x_querry, l, x_block, e_p, task_id):
    B = x_querry.shape[0]
    pool = e_p.reshape(e_p.shape[0] * e_p.shape[1], _EMB_D)
    l_i = jnp.asarray(l, jnp.int32)
    valid = ((l_i >= 0) & (l_i < 12)).astype(jnp.int32)
    scalars = jnp.stack([jnp.asarray(task_id, jnp.int32), valid])
    P, xc = pl.pallas_call(
        _body,
        grid_spec=pltpu.PrefetchScalarGridSpec(
            num_scalar_prefetch=1,
            grid=(pl.cdiv(B, _BB),),
            in_specs=[
                pl.BlockSpec((pool.shape[0], _EMB_D), lambda i, s: (0, 0)),
                pl.BlockSpec((_BB, _EMB_D), lambda i, s: (i, 0)),
            ],
            out_specs=[
                pl.BlockSpec((_BB, None, _EMB_D), lambda i, s: (i, 0, 0)),
                pl.BlockSpec((_BB, _EMB_D), lambda i, s: (i, 0)),
            ],
        ),
        out_shape=[
            jax.ShapeDtypeStruct((B, e_p.shape[1], _EMB_D), jnp.float32),
            jax.ShapeDtypeStruct((B, _EMB_D), jnp.float32),
        ],
    )(scalars, pool, x_block)
    return (P, xc)
